# baseline (device time: 15786 ns/iter reference)
import jax
import jax.numpy as jnp
from jax import lax
from jax.experimental import pallas as pl
from jax.experimental.pallas import tpu as pltpu

N_DEV = 8
ROWS = 512
CHUNK = ROWS // N_DEV
E_LOC = 2


def kernel(x, router_W, route_idx, expert_W, shared_W):
    d_model = x.shape[1]
    d_hidden = expert_W.shape[2]
    n_exp = router_W.shape[1]

    def body(x_ref, rw_ref, idx_ref, ew_ref, sw_ref, out_ref,
             send_ref, recv_ref, d0_ref, xw1_ref, send_sems, recv_sems):
        my = lax.axis_index("i")

        barrier_sem = pltpu.get_barrier_semaphore()
        for d in range(N_DEV):
            @pl.when(d != my)
            def _():
                pl.semaphore_signal(
                    barrier_sem, inc=1,
                    device_id=(d,), device_id_type=pl.DeviceIdType.MESH,
                )
        pl.semaphore_wait(barrier_sem, N_DEV - 1)

        x_all = x_ref[...]
        scores = jnp.dot(x_all, rw_ref[...], preferred_element_type=jnp.float32)
        s_max = jnp.max(scores, axis=1, keepdims=True)
        e = jnp.exp(scores - s_max)
        probs = e / jnp.sum(e, axis=1, keepdims=True)
        idx = idx_ref[...]
        eids = lax.broadcasted_iota(jnp.int32, (ROWS, n_exp), 1)
        routed_p = jnp.sum(
            probs * (eids == idx).astype(jnp.float32), axis=1, keepdims=True
        )

        w0 = routed_p * (idx == my * E_LOC).astype(jnp.float32)
        w1 = routed_p * (idx == my * E_LOC + 1).astype(jnp.float32)

        d0_ref[...] = jnp.dot(
            (w0 * x_all).astype(jnp.bfloat16),
            ew_ref[0].astype(jnp.bfloat16),
            preferred_element_type=jnp.float32,
        )
        xw1_ref[...] = (w1 * x_all).astype(jnp.bfloat16)
        ew1 = ew_ref[1].astype(jnp.bfloat16)

        for k in range(N_DEV - 1):
            t = (my + 1 + k) % N_DEV
            rows = pl.ds(t * CHUNK, CHUNK)
            chunk = (
                jnp.dot(xw1_ref[rows, :], ew1,
                        preferred_element_type=jnp.float32)
                + d0_ref[rows, :]
            )
            send_ref[k] = chunk.astype(jnp.bfloat16)
            rdma = pltpu.make_async_remote_copy(
                src_ref=send_ref.at[k],
                dst_ref=recv_ref.at[my],
                send_sem=send_sems.at[k],
                recv_sem=recv_sems.at[my],
                device_id=(t,),
                device_id_type=pl.DeviceIdType.MESH,
            )
            rdma.start()

        rows_my = pl.ds(my * CHUNK, CHUNK)
        own = (
            jnp.dot(xw1_ref[rows_my, :], ew1,
                    preferred_element_type=jnp.float32)
            + d0_ref[rows_my, :]
        )
        shared_chunk = jnp.dot(
            x_ref[rows_my, :].astype(jnp.bfloat16),
            sw_ref[...].astype(jnp.bfloat16),
            preferred_element_type=jnp.float32,
        )
        out_ref[...] = shared_chunk + own

        for s in range(N_DEV):
            @pl.when(s != my)
            def _():
                recv = pltpu.make_async_remote_copy(
                    src_ref=recv_ref.at[s],
                    dst_ref=recv_ref.at[s],
                    send_sem=send_sems.at[0],
                    recv_sem=recv_sems.at[s],
                    device_id=(s,),
                    device_id_type=pl.DeviceIdType.MESH,
                )
                recv.wait_recv()
                out_ref[...] += recv_ref[s].astype(jnp.float32)

        for k in range(N_DEV - 1):
            send = pltpu.make_async_remote_copy(
                src_ref=send_ref.at[k],
                dst_ref=send_ref.at[k],
                send_sem=send_sems.at[k],
                recv_sem=recv_sems.at[0],
                device_id=(0,),
                device_id_type=pl.DeviceIdType.MESH,
            )
            send.wait_send()

    return pl.pallas_call(
        body,
        out_shape=jax.ShapeDtypeStruct((CHUNK, d_hidden), jnp.float32),
        in_specs=[pl.BlockSpec(memory_space=pltpu.VMEM)] * 5,
        out_specs=pl.BlockSpec(memory_space=pltpu.VMEM),
        scratch_shapes=[
            pltpu.VMEM((N_DEV - 1, CHUNK, d_hidden), jnp.bfloat16),
            pltpu.VMEM((N_DEV, CHUNK, d_hidden), jnp.bfloat16),
            pltpu.VMEM((ROWS, d_hidden), jnp.float32),
            pltpu.VMEM((ROWS, d_model), jnp.bfloat16),
            pltpu.SemaphoreType.DMA((N_DEV - 1,)),
            pltpu.SemaphoreType.DMA((N_DEV,)),
        ],
        compiler_params=pltpu.CompilerParams(collective_id=0),
    )(x, router_W, route_idx, expert_W, shared_W)


# device time: 15710 ns/iter; 1.0048x vs baseline; 1.0048x over previous
import os

import jax
import jax.numpy as jnp
from jax import lax
from jax.experimental import pallas as pl
from jax.experimental.pallas import tpu as pltpu

_ABLATE = os.environ.get("ABLATE", "")
_DO_COMPUTE = _ABLATE != "comm"
_DO_COMM = _ABLATE != "compute"

N_DEV = 8
ROWS = 512
CHUNK = ROWS // N_DEV
E_LOC = 2


def kernel(x, router_W, route_idx, expert_W, shared_W):
    d_model = x.shape[1]
    d_hidden = expert_W.shape[2]
    n_exp = router_W.shape[1]

    def body(x_hbm, rw_ref, idx_ref, ew_hbm, sw_hbm, out_ref,
             send_ref, recv_ref, d0_ref, xw1_ref, x_ref, ew_ref, sw_ref,
             send_sems, recv_sems, load_sems):
        my = lax.axis_index("i")

        cp_x = pltpu.make_async_copy(x_hbm, x_ref, load_sems.at[0])
        cp_ew = pltpu.make_async_copy(ew_hbm, ew_ref, load_sems.at[1])
        cp_sw = pltpu.make_async_copy(sw_hbm, sw_ref, load_sems.at[2])
        cp_x.start()
        cp_ew.start()
        cp_sw.start()

        if _DO_COMM:
            barrier_sem = pltpu.get_barrier_semaphore()
            for d in range(N_DEV):
                @pl.when(d != my)
                def _():
                    pl.semaphore_signal(
                        barrier_sem, inc=1,
                        device_id=(d,), device_id_type=pl.DeviceIdType.MESH,
                    )
            pl.semaphore_wait(barrier_sem, N_DEV - 1)

        if not _DO_COMPUTE:
            out_ref[...] = jnp.zeros((CHUNK, d_hidden), jnp.float32)
            ew1 = None
        if _DO_COMPUTE:
            cp_x.wait()
            x_all = x_ref[...]
            scores = jnp.dot(
                x_all, rw_ref[...], preferred_element_type=jnp.float32
            )
            s_max = jnp.max(scores, axis=1, keepdims=True)
            e = jnp.exp(scores - s_max)
            probs = e / jnp.sum(e, axis=1, keepdims=True)
            idx = idx_ref[...]
            eids = lax.broadcasted_iota(jnp.int32, (ROWS, n_exp), 1)
            routed_p = jnp.sum(
                probs * (eids == idx).astype(jnp.float32), axis=1, keepdims=True
            )

            w0 = routed_p * (idx == my * E_LOC).astype(jnp.float32)
            w1 = routed_p * (idx == my * E_LOC + 1).astype(jnp.float32)

            cp_ew.wait()
            d0_ref[...] = jnp.dot(
                (w0 * x_all).astype(jnp.bfloat16),
                ew_ref[0].astype(jnp.bfloat16),
                preferred_element_type=jnp.float32,
            )
            xw1_ref[...] = (w1 * x_all).astype(jnp.bfloat16)
            ew1 = ew_ref[1].astype(jnp.bfloat16)

        for k in range(N_DEV - 1):
            t = (my + 1 + k) % N_DEV
            if _DO_COMPUTE:
                rows = pl.ds(t * CHUNK, CHUNK)
                chunk = (
                    jnp.dot(xw1_ref[rows, :], ew1,
                            preferred_element_type=jnp.float32)
                    + d0_ref[rows, :]
                )
                send_ref[k] = chunk.astype(jnp.bfloat16)
            if not _DO_COMM:
                continue
            rdma = pltpu.make_async_remote_copy(
                src_ref=send_ref.at[k],
                dst_ref=recv_ref.at[my],
                send_sem=send_sems.at[k],
                recv_sem=recv_sems.at[my],
                device_id=(t,),
                device_id_type=pl.DeviceIdType.MESH,
            )
            rdma.start()

        if _DO_COMPUTE:
            rows_my = pl.ds(my * CHUNK, CHUNK)
            own = (
                jnp.dot(xw1_ref[rows_my, :], ew1,
                        preferred_element_type=jnp.float32)
                + d0_ref[rows_my, :]
            )
            cp_sw.wait()
            shared_chunk = jnp.dot(
                x_ref[rows_my, :].astype(jnp.bfloat16),
                sw_ref[...].astype(jnp.bfloat16),
                preferred_element_type=jnp.float32,
            )
            out_ref[...] = shared_chunk + own

        for s in range(N_DEV if _DO_COMM else 0):
            @pl.when(s != my)
            def _():
                recv = pltpu.make_async_remote_copy(
                    src_ref=recv_ref.at[s],
                    dst_ref=recv_ref.at[s],
                    send_sem=send_sems.at[0],
                    recv_sem=recv_sems.at[s],
                    device_id=(s,),
                    device_id_type=pl.DeviceIdType.MESH,
                )
                recv.wait_recv()
                out_ref[...] += recv_ref[s].astype(jnp.float32)

        if not _DO_COMPUTE:
            cp_x.wait()
            cp_ew.wait()
            cp_sw.wait()

        for k in range(N_DEV - 1 if _DO_COMM else 0):
            send = pltpu.make_async_remote_copy(
                src_ref=send_ref.at[k],
                dst_ref=send_ref.at[k],
                send_sem=send_sems.at[k],
                recv_sem=recv_sems.at[0],
                device_id=(0,),
                device_id_type=pl.DeviceIdType.MESH,
            )
            send.wait_send()

    return pl.pallas_call(
        body,
        out_shape=jax.ShapeDtypeStruct((CHUNK, d_hidden), jnp.float32),
        in_specs=[
            pl.BlockSpec(memory_space=pl.ANY),
            pl.BlockSpec(memory_space=pltpu.VMEM),
            pl.BlockSpec(memory_space=pltpu.VMEM),
            pl.BlockSpec(memory_space=pl.ANY),
            pl.BlockSpec(memory_space=pl.ANY),
        ],
        out_specs=pl.BlockSpec(memory_space=pltpu.VMEM),
        scratch_shapes=[
            pltpu.VMEM((N_DEV - 1, CHUNK, d_hidden), jnp.bfloat16),
            pltpu.VMEM((N_DEV, CHUNK, d_hidden), jnp.bfloat16),
            pltpu.VMEM((ROWS, d_hidden), jnp.float32),
            pltpu.VMEM((ROWS, d_model), jnp.bfloat16),
            pltpu.VMEM((ROWS, d_model), jnp.float32),
            pltpu.VMEM((E_LOC, d_model, d_hidden), jnp.float32),
            pltpu.VMEM((d_model, d_hidden), jnp.float32),
            pltpu.SemaphoreType.DMA((N_DEV - 1,)),
            pltpu.SemaphoreType.DMA((N_DEV,)),
            pltpu.SemaphoreType.DMA((3,)),
        ],
        compiler_params=pltpu.CompilerParams(
            collective_id=0 if _DO_COMM else None
        ),
    )(x, router_W, route_idx, expert_W, shared_W)


# device time: 12041 ns/iter; 1.3110x vs baseline; 1.3047x over previous
import os

import jax
import jax.numpy as jnp
from jax import lax
from jax.experimental import pallas as pl
from jax.experimental.pallas import tpu as pltpu

_ABLATE = os.environ.get("ABLATE", "")
_DO_COMPUTE = _ABLATE != "comm"
_DO_COMM = _ABLATE != "compute"

N_DEV = 8
ROWS = 512
CHUNK = ROWS // N_DEV
E_LOC = 2


def kernel(x, router_W, route_idx, expert_W, shared_W):
    d_model = x.shape[1]
    d_hidden = expert_W.shape[2]
    n_exp = router_W.shape[1]

    def body(x_hbm, rw_ref, idx_ref, ew_hbm, sw_hbm, out_ref,
             send_ref, recv_ref, d0_ref, xw1_ref, x_ref, ew_ref, sw_ref,
             send_sems, recv_sems, load_sems):
        my = lax.axis_index("i")

        cp_x = pltpu.make_async_copy(x_hbm, x_ref, load_sems.at[0])
        cp_ew = pltpu.make_async_copy(ew_hbm, ew_ref, load_sems.at[1])
        cp_sw = pltpu.make_async_copy(sw_hbm, sw_ref, load_sems.at[2])
        cp_x.start()
        cp_ew.start()
        cp_sw.start()

        if _DO_COMM:
            barrier_sem = pltpu.get_barrier_semaphore()
            for d in range(N_DEV):
                @pl.when(d != my)
                def _():
                    pl.semaphore_signal(
                        barrier_sem, inc=1,
                        device_id=(d,), device_id_type=pl.DeviceIdType.MESH,
                    )
            pl.semaphore_wait(barrier_sem, N_DEV - 1)

        if not _DO_COMPUTE:
            out_ref[...] = jnp.zeros((CHUNK, d_hidden), jnp.float32)
            ew1 = None
        if _DO_COMPUTE:
            cp_x.wait()
            x_all = x_ref[...]
            scores = jnp.dot(
                x_all, rw_ref[...], preferred_element_type=jnp.float32
            )
            s_max = jnp.max(scores, axis=1, keepdims=True)
            e = jnp.exp(scores - s_max)
            probs = e / jnp.sum(e, axis=1, keepdims=True)
            idx = idx_ref[...]
            eids = lax.broadcasted_iota(jnp.int32, (ROWS, n_exp), 1)
            routed_p = jnp.sum(
                probs * (eids == idx).astype(jnp.float32), axis=1, keepdims=True
            )

            w0 = routed_p * (idx == my * E_LOC).astype(jnp.float32)
            w1 = routed_p * (idx == my * E_LOC + 1).astype(jnp.float32)

            cp_ew.wait()
            d0_ref[...] = jnp.dot(
                (w0 * x_all).astype(jnp.bfloat16),
                ew_ref[0].astype(jnp.bfloat16),
                preferred_element_type=jnp.float32,
            )
            xw1_ref[...] = (w1 * x_all).astype(jnp.bfloat16)
            ew1 = ew_ref[1].astype(jnp.bfloat16)

        for k in range(N_DEV - 1):
            t = (my + 1 + k) % N_DEV
            if _DO_COMPUTE:
                rows = pl.ds(t * CHUNK, CHUNK)
                chunk = (
                    jnp.dot(xw1_ref[rows, :], ew1,
                            preferred_element_type=jnp.float32)
                    + d0_ref[rows, :]
                )
                send_ref[k] = chunk.astype(jnp.bfloat16)
            if not _DO_COMM:
                continue
            rdma = pltpu.make_async_remote_copy(
                src_ref=send_ref.at[k],
                dst_ref=recv_ref.at[my],
                send_sem=send_sems.at[k],
                recv_sem=recv_sems.at[my],
                device_id=(t,),
                device_id_type=pl.DeviceIdType.MESH,
            )
            rdma.start()

        if _DO_COMPUTE:
            rows_my = pl.ds(my * CHUNK, CHUNK)
            own = (
                jnp.dot(xw1_ref[rows_my, :], ew1,
                        preferred_element_type=jnp.float32)
                + d0_ref[rows_my, :]
            )
            cp_sw.wait()
            shared_chunk = jnp.dot(
                x_ref[rows_my, :].astype(jnp.bfloat16),
                sw_ref[...].astype(jnp.bfloat16),
                preferred_element_type=jnp.float32,
            )
            out_ref[...] = shared_chunk + own

        for s in range(N_DEV if _DO_COMM else 0):
            @pl.when(s != my)
            def _():
                recv = pltpu.make_async_remote_copy(
                    src_ref=recv_ref.at[s],
                    dst_ref=recv_ref.at[s],
                    send_sem=send_sems.at[0],
                    recv_sem=recv_sems.at[s],
                    device_id=(s,),
                    device_id_type=pl.DeviceIdType.MESH,
                )
                recv.wait_recv()
                out_ref[...] += recv_ref[s].astype(jnp.float32)

        if not _DO_COMPUTE:
            cp_x.wait()
            cp_ew.wait()
            cp_sw.wait()

        for k in range(N_DEV - 1 if _DO_COMM else 0):
            send = pltpu.make_async_remote_copy(
                src_ref=send_ref.at[k],
                dst_ref=send_ref.at[k],
                send_sem=send_sems.at[k],
                recv_sem=recv_sems.at[0],
                device_id=(0,),
                device_id_type=pl.DeviceIdType.MESH,
            )
            send.wait_send()

    return pl.pallas_call(
        body,
        out_shape=jax.ShapeDtypeStruct((CHUNK, d_hidden), jnp.float32),
        in_specs=[
            pl.BlockSpec(memory_space=pl.ANY),
            pl.BlockSpec(memory_space=pltpu.VMEM),
            pl.BlockSpec(memory_space=pltpu.VMEM),
            pl.BlockSpec(memory_space=pl.ANY),
            pl.BlockSpec(memory_space=pl.ANY),
        ],
        out_specs=pl.BlockSpec(memory_space=pltpu.VMEM),
        scratch_shapes=[
            pltpu.VMEM((N_DEV - 1, CHUNK, d_hidden), jnp.bfloat16),
            pltpu.VMEM((N_DEV, CHUNK, d_hidden), jnp.bfloat16),
            pltpu.VMEM((ROWS, d_hidden), jnp.float32),
            pltpu.VMEM((ROWS, d_model), jnp.bfloat16),
            pltpu.VMEM((ROWS, d_model), jnp.float32),
            pltpu.VMEM((E_LOC, d_model, d_hidden), jnp.float32),
            pltpu.VMEM((d_model, d_hidden), jnp.float32),
            pltpu.SemaphoreType.DMA((N_DEV - 1,)),
            pltpu.SemaphoreType.DMA((N_DEV,)),
            pltpu.SemaphoreType.DMA((3,)),
        ],
        compiler_params=pltpu.CompilerParams(
            collective_id=0 if _DO_COMM else None
        ),
    )(
        pltpu.with_memory_space_constraint(x, pltpu.MemorySpace.HBM),
        router_W,
        route_idx,
        pltpu.with_memory_space_constraint(expert_W, pltpu.MemorySpace.HBM),
        pltpu.with_memory_space_constraint(shared_W, pltpu.MemorySpace.HBM),
    )
